# Initial kernel scaffold; baseline (speedup 1.0000x reference)
import functools

import jax
import jax.numpy as jnp
from jax import lax
from jax.experimental import pallas as pl
from jax.experimental.pallas import tpu as pltpu
from jax.experimental.pallas import tpu_sc as plsc

VOCAB = 100000
EMB = 64
B_TOTAL = 4096 * 50
CHUNK = 128
NC, NS = 2, 16
NW = NC * NS
ROWS_PER_W = B_TOTAL // (NW * CHUNK)


def _sc_gather(table, idx2d):
    mesh = plsc.VectorSubcoreMesh(
        core_axis_name="c", subcore_axis_name="s",
        num_cores=NC, num_subcores=NS)

    @functools.partial(
        pl.kernel,
        out_type=jax.ShapeDtypeStruct((B_TOTAL, EMB), jnp.float32),
        mesh=mesh,
        scratch_types=[
            pltpu.VMEM((ROWS_PER_W, CHUNK), jnp.int32),
            pltpu.VMEM((CHUNK, EMB), jnp.float32),
            pltpu.SemaphoreType.DMA,
        ],
    )
    def k(table_hbm, idx_hbm, out_hbm, idx_v, rows_v, gsem):
        wid = lax.axis_index("s") * NC + lax.axis_index("c")
        pltpu.sync_copy(idx_hbm.at[pl.ds(wid * ROWS_PER_W, ROWS_PER_W)], idx_v)

        def step(c, carry):
            pltpu.async_copy(table_hbm.at[idx_v.at[c]], rows_v, gsem).wait()
            out_off = (wid * ROWS_PER_W + c) * CHUNK
            pltpu.sync_copy(rows_v, out_hbm.at[pl.ds(out_off, CHUNK)])
            return carry

        lax.fori_loop(0, ROWS_PER_W, step, 0)

    return k(table, idx2d)


def kernel(seq, W):
    idx2d = seq.reshape(B_TOTAL // CHUNK, CHUNK).astype(jnp.int32)
    out = _sc_gather(W, idx2d)
    return out.reshape(seq.shape[0], seq.shape[1], EMB)


# speedup vs baseline: 4.0920x; 4.0920x over previous
import functools

import jax
import jax.numpy as jnp
from jax import lax
from jax.experimental import pallas as pl
from jax.experimental.pallas import tpu as pltpu
from jax.experimental.pallas import tpu_sc as plsc

VOCAB = 100000
EMB = 64
B_TOTAL = 4096 * 50
CHUNK = 128
NC, NS = 2, 16
NW = NC * NS
ROWS_PER_W = B_TOTAL // (NW * CHUNK)


def _sc_gather(table, idx2d):
    mesh = plsc.VectorSubcoreMesh(
        core_axis_name="c", subcore_axis_name="s",
        num_cores=NC, num_subcores=NS)

    @functools.partial(
        pl.kernel,
        out_type=jax.ShapeDtypeStruct((B_TOTAL, EMB), jnp.float32),
        mesh=mesh,
        scratch_types=[
            pltpu.VMEM((ROWS_PER_W, CHUNK), jnp.int32),
            pltpu.VMEM((CHUNK, EMB), jnp.float32),
            pltpu.SemaphoreType.DMA,
        ],
        compiler_params=pltpu.CompilerParams(use_tc_tiling_on_sc=False),
    )
    def k(table_hbm, idx_hbm, out_hbm, idx_v, rows_v, gsem):
        wid = lax.axis_index("s") * NC + lax.axis_index("c")
        pltpu.sync_copy(idx_hbm.at[wid], idx_v)

        def step(c, carry):
            pltpu.async_copy(table_hbm.at[idx_v.at[c]], rows_v, gsem).wait()
            out_off = (wid * ROWS_PER_W + c) * CHUNK
            pltpu.sync_copy(rows_v, out_hbm.at[pl.ds(out_off, CHUNK)])
            return carry

        lax.fori_loop(0, ROWS_PER_W, step, 0)

    return k(table, idx2d)


def kernel(seq, W):
    idx2d = seq.reshape(NW, ROWS_PER_W, CHUNK).astype(jnp.int32)
    out = _sc_gather(W, idx2d)
    return out.reshape(seq.shape[0], seq.shape[1], EMB)


# speedup vs baseline: 4.6805x; 1.1438x over previous
import functools

import jax
import jax.numpy as jnp
from jax import lax
from jax.experimental import pallas as pl
from jax.experimental.pallas import tpu as pltpu
from jax.experimental.pallas import tpu_sc as plsc

VOCAB = 100000
EMB = 64
B_TOTAL = 4096 * 50
CHUNK = 128
NC, NS = 2, 16
NW = NC * NS
ROWS_PER_W = B_TOTAL // (NW * CHUNK)
NBUF = 5


def _sc_gather(table, idx2d):
    mesh = plsc.VectorSubcoreMesh(
        core_axis_name="c", subcore_axis_name="s",
        num_cores=NC, num_subcores=NS)

    @functools.partial(
        pl.kernel,
        out_type=jax.ShapeDtypeStruct((B_TOTAL, EMB), jnp.float32),
        mesh=mesh,
        scratch_types=[
            pltpu.VMEM((ROWS_PER_W, CHUNK), jnp.int32),
            [pltpu.VMEM((CHUNK, EMB), jnp.float32) for _ in range(NBUF)],
            [pltpu.SemaphoreType.DMA for _ in range(NBUF)],
        ],
        compiler_params=pltpu.CompilerParams(use_tc_tiling_on_sc=False),
    )
    def k(table_hbm, idx_hbm, out_hbm, idx_v, rows, gsems):
        wid = lax.axis_index("s") * NC + lax.axis_index("c")
        pltpu.sync_copy(idx_hbm.at[wid], idx_v)

        for b in range(NBUF):
            pltpu.async_copy(table_hbm.at[idx_v.at[b]], rows[b], gsems[b])

        def outer(g, carry):
            for b in range(NBUF):
                c = g * NBUF + b
                pltpu.make_async_copy(
                    table_hbm.at[idx_v.at[c]], rows[b], gsems[b]).wait()
                out_off = (wid * ROWS_PER_W + c) * CHUNK
                pltpu.sync_copy(rows[b], out_hbm.at[pl.ds(out_off, CHUNK)])
                nxt = c + NBUF

                @pl.when(nxt < ROWS_PER_W)
                def _():
                    pltpu.async_copy(
                        table_hbm.at[idx_v.at[nxt]], rows[b], gsems[b])
            return carry

        lax.fori_loop(0, ROWS_PER_W // NBUF, outer, 0)

    return k(table, idx2d)


def kernel(seq, W):
    idx2d = seq.reshape(NW, ROWS_PER_W, CHUNK).astype(jnp.int32)
    out = _sc_gather(W, idx2d)
    return out.reshape(seq.shape[0], seq.shape[1], EMB)


# speedup vs baseline: 5.1055x; 1.0908x over previous
import functools

import jax
import jax.numpy as jnp
from jax import lax
from jax.experimental import pallas as pl
from jax.experimental.pallas import tpu as pltpu
from jax.experimental.pallas import tpu_sc as plsc

VOCAB = 100000
EMB = 64
BATCH = 4096
SEQ = 50
LANES = 16
NC, NS = 2, 16
NW = NC * NS
B_PER_W = BATCH // NW
NBUF = 4


def _sc_lookup(table_pad, seq):
    mesh = plsc.VectorSubcoreMesh(
        core_axis_name="c", subcore_axis_name="s",
        num_cores=NC, num_subcores=NS)

    @functools.partial(
        pl.kernel,
        out_type=jax.ShapeDtypeStruct((BATCH, SEQ, EMB), jnp.float32),
        mesh=mesh,
        scratch_types=[
            pltpu.VMEM((B_PER_W, SEQ), jnp.int32),
            [pltpu.VMEM((SEQ, 2 * EMB), jnp.float32) for _ in range(NBUF)],
            [pltpu.VMEM((SEQ, EMB), jnp.float32) for _ in range(NBUF)],
            [pltpu.SemaphoreType.DMA for _ in range(NBUF)],
            [pltpu.SemaphoreType.DMA for _ in range(NBUF)],
        ],
        compiler_params=pltpu.CompilerParams(use_tc_tiling_on_sc=True),
    )
    def k(table_hbm, seq_hbm, out_hbm, idx_v, rows, stages, gsems, ssems):
        wid = lax.axis_index("s") * NC + lax.axis_index("c")
        b0 = wid * B_PER_W
        pltpu.sync_copy(seq_hbm.at[pl.ds(b0, B_PER_W)], idx_v)

        for b in range(NBUF):
            pltpu.async_copy(table_hbm.at[idx_v.at[b]], rows[b], gsems[b])

        def repack(rbuf, sbuf):
            def per_row(r, carry):
                for l in range(EMB // LANES):
                    sbuf[r, pl.ds(l * LANES, LANES)] = (
                        rbuf[r, pl.ds(l * LANES, LANES)])
                return carry
            lax.fori_loop(0, SEQ, per_row, 0)

        def outer(g, carry):
            for b in range(NBUF):
                c = g * NBUF + b
                pltpu.make_async_copy(
                    table_hbm.at[idx_v.at[c]], rows[b], gsems[b]).wait()

                @pl.when(g > 0)
                def _():
                    pltpu.make_async_copy(
                        stages[b], out_hbm.at[b0 + c - NBUF], ssems[b]).wait()

                repack(rows[b], stages[b])
                pltpu.async_copy(stages[b], out_hbm.at[b0 + c], ssems[b])
                nxt = c + NBUF

                @pl.when(nxt < B_PER_W)
                def _():
                    pltpu.async_copy(
                        table_hbm.at[idx_v.at[nxt]], rows[b], gsems[b])
            return carry

        lax.fori_loop(0, B_PER_W // NBUF, outer, 0)
        for b in range(NBUF):
            pltpu.make_async_copy(
                stages[b], out_hbm.at[b0 + B_PER_W - NBUF + b], ssems[b]).wait()

    return k(table_pad, seq)


def kernel(seq, W):
    table_pad = jnp.pad(W, ((0, 0), (0, 2 * EMB - W.shape[1])))
    return _sc_lookup(table_pad, seq.astype(jnp.int32))
